# baseline (speedup 1.0000x reference)
import jax
import jax.numpy as jnp
from jax import lax
from jax.experimental import pallas as pl
from jax.experimental.pallas import tpu as pltpu
from jax.experimental.pallas import tpu_sc as plsc

NC, NS = 2, 16
NW = NC * NS
CHUNK = 128
K = 8


def _emb_body(idx_hbm, table_hbm, out_hbm, idx_v, rows_v, gsem, osem):
    wid = lax.axis_index("s") * NC + lax.axis_index("c")
    n_chunks = idx_hbm.shape[1]
    base = wid * (n_chunks * CHUNK)

    pltpu.sync_copy(idx_hbm.at[wid], idx_v)

    def group(g, carry):
        gathers = []
        for i in range(K):
            j = g * K + i
            gathers.append(
                pltpu.async_copy(table_hbm.at[idx_v.at[j]], rows_v.at[i],
                                 gsem.at[i]))
        writes = []
        for i in range(K):
            j = g * K + i
            gathers[i].wait()
            writes.append(
                pltpu.async_copy(rows_v.at[i],
                                 out_hbm.at[pl.ds(base + j * CHUNK, CHUNK)],
                                 osem.at[i]))
        for i in range(K):
            writes[i].wait()
        return carry

    lax.fori_loop(0, n_chunks // K, group, 0)


def kernel(indices, table):
    B, H = indices.shape
    D = table.shape[1]
    total = B * H
    per_w = total // NW
    n_chunks = per_w // CHUNK
    idx = indices.reshape(NW, n_chunks, CHUNK).astype(jnp.int32)

    run = pl.kernel(
        _emb_body,
        out_type=jax.ShapeDtypeStruct((total, D), jnp.float32),
        mesh=plsc.VectorSubcoreMesh(core_axis_name="c", subcore_axis_name="s"),
        compiler_params=pltpu.CompilerParams(use_tc_tiling_on_sc=False),
        scratch_types=[
            pltpu.VMEM((n_chunks, CHUNK), jnp.int32),
            pltpu.VMEM((K, CHUNK, D), jnp.float32),
            pltpu.SemaphoreType.DMA((K,)),
            pltpu.SemaphoreType.DMA((K,)),
        ],
    )
    out = run(idx, table)
    return out.reshape(B, H, D)


# speedup vs baseline: 1.0010x; 1.0010x over previous
import jax
import jax.numpy as jnp
from jax import lax
from jax.experimental import pallas as pl
from jax.experimental.pallas import tpu as pltpu
from jax.experimental.pallas import tpu_sc as plsc

NC, NS = 2, 16
NW = NC * NS
CHUNK = 128
K = 8


L = 4


def _emb_body(idx_hbm, table_hbm, out_hbm, idx_v, rows_v, gsem, osem):
    wid = lax.axis_index("s") * NC + lax.axis_index("c")
    n_chunks = idx_hbm.shape[1]
    base = wid * (n_chunks * CHUNK)

    pltpu.sync_copy(idx_hbm.at[wid], idx_v)

    def start_gather(j, b):
        pltpu.async_copy(table_hbm.at[idx_v.at[j]], rows_v.at[b], gsem.at[b])

    def wait_gather(b):
        pltpu.make_async_copy(table_hbm.at[idx_v.at[0]], rows_v.at[b],
                              gsem.at[b]).wait()

    def start_out(j, b):
        pltpu.async_copy(rows_v.at[b],
                         out_hbm.at[pl.ds(base + j * CHUNK, CHUNK)],
                         osem.at[b])

    def wait_out(b):
        pltpu.make_async_copy(rows_v.at[b], out_hbm.at[pl.ds(base, CHUNK)],
                              osem.at[b]).wait()

    for j in range(L):
        start_gather(j, j)
    for j in range(K - L):
        start_gather(j + L, j + L)
        wait_gather(j)
        start_out(j, j)

    def steady(j, carry):
        bg = (j + L) % K
        wait_out(bg)
        start_gather(j + L, bg)
        b = j % K
        wait_gather(b)
        start_out(j, b)
        return carry

    lax.fori_loop(K - L, n_chunks - L, steady, 0)

    for j in range(n_chunks - L, n_chunks):
        wait_gather(j % K)
        start_out(j, j % K)
    for j in range(n_chunks - K, n_chunks):
        wait_out(j % K)


def kernel(indices, table):
    B, H = indices.shape
    D = table.shape[1]
    total = B * H
    per_w = total // NW
    n_chunks = per_w // CHUNK
    idx = indices.reshape(NW, n_chunks, CHUNK).astype(jnp.int32)

    run = pl.kernel(
        _emb_body,
        out_type=jax.ShapeDtypeStruct((total, D), jnp.float32),
        mesh=plsc.VectorSubcoreMesh(core_axis_name="c", subcore_axis_name="s"),
        compiler_params=pltpu.CompilerParams(use_tc_tiling_on_sc=False),
        scratch_types=[
            pltpu.VMEM((n_chunks, CHUNK), jnp.int32),
            pltpu.VMEM((K, CHUNK, D), jnp.float32),
            pltpu.SemaphoreType.DMA((K,)),
            pltpu.SemaphoreType.DMA((K,)),
        ],
    )
    out = run(idx, table)
    return out.reshape(B, H, D)
